# Initial kernel scaffold; baseline (speedup 1.0000x reference)
import jax
import jax.numpy as jnp
from jax.experimental import pallas as pl


def kernel(sequence, token_type_ids, position_embeddings, token_type_embeddings, ln_gamma, ln_beta):
    raise NotImplementedError("write your pallas kernel here")


# speedup vs baseline: 4.3328x; 4.3328x over previous
import jax
import jax.numpy as jnp
from jax.experimental import pallas as pl

EPS = 1e-12


def _body(seq_ref, pos_ref, ids_ref, tt_ref, g_ref, b_ref, out_ref):
    x = seq_ref[0] + pos_ref[...]
    ids = ids_ref[0]
    t0 = tt_ref[0:1, :]
    t1 = tt_ref[1:2, :]
    x = x + t0 + ids * (t1 - t0)
    u = jnp.mean(x, axis=1, keepdims=True)
    xc = x - u
    var = jnp.mean(xc * xc, axis=1, keepdims=True)
    normed = xc / jnp.sqrt(var + EPS)
    out_ref[0] = normed * g_ref[...] + b_ref[...]


def kernel(sequence, token_type_ids, position_embeddings, token_type_embeddings, ln_gamma, ln_beta):
    B, S, H = sequence.shape
    SBLK = 512
    nS = S // SBLK
    ids_col = token_type_ids.astype(jnp.float32).reshape(B, S, 1)
    g2 = ln_gamma.reshape(1, H)
    b2 = ln_beta.reshape(1, H)
    return pl.pallas_call(
        _body,
        grid=(nS, B),
        in_specs=[
            pl.BlockSpec((1, SBLK, H), lambda j, b: (b, j, 0)),
            pl.BlockSpec((SBLK, H), lambda j, b: (j, 0)),
            pl.BlockSpec((1, SBLK, 1), lambda j, b: (b, j, 0)),
            pl.BlockSpec((2, H), lambda j, b: (0, 0)),
            pl.BlockSpec((1, H), lambda j, b: (0, 0)),
            pl.BlockSpec((1, H), lambda j, b: (0, 0)),
        ],
        out_specs=pl.BlockSpec((1, SBLK, H), lambda j, b: (b, j, 0)),
        out_shape=jax.ShapeDtypeStruct((B, S, H), jnp.float32),
    )(sequence, position_embeddings, ids_col, token_type_embeddings, g2, b2)
